# baseline (speedup 1.0000x reference)
import jax
import jax.numpy as jnp
from jax.experimental import pallas as pl
from jax.experimental.pallas import tpu as pltpu

_NE = 1024
_ED = 64
_BETA = 0.25
_TOK = 1024


def _conv_block(xb, w1b, b1b, w2, b2, pq):
    xb = xb.reshape(192, 512).astype(jnp.bfloat16)
    xg = [jnp.dot(xb[:, 128 * g:128 * g + 128], pq,
                  preferred_element_type=jnp.float32).astype(jnp.bfloat16)
          for g in range(4)]
    xs = jnp.concatenate([
        jnp.concatenate([xg[g][:, 32 * dx:32 * dx + 32] for g in range(4)],
                        axis=1)
        for dx in range(4)], axis=0)
    ht = jnp.dot(w1b, xs, preferred_element_type=jnp.float32) + b1b
    ht = jnp.maximum(ht, 0.0).astype(jnp.bfloat16)
    htp = jnp.dot(ht, pq,
                  preferred_element_type=jnp.float32).astype(jnp.bfloat16)
    ball = jnp.concatenate([
        jnp.concatenate([htp[256 * i:256 * i + 256, 32 * d:32 * d + 32]
                         for d in range(4)], axis=0)
        for i in range(4)], axis=1)
    h2t = jnp.dot(w2, ball, preferred_element_type=jnp.float32) + b2
    return jnp.transpose(h2t)


def _shift_tokens(hr, dy, dx):
    s = 32 * dy + dx
    if s > 0:
        sh = jnp.concatenate(
            [hr[s:], jnp.zeros((s, hr.shape[1]), jnp.float32)], axis=0)
    elif s < 0:
        sh = jnp.concatenate(
            [jnp.zeros((-s, hr.shape[1]), jnp.float32), hr[:1024 + s]], axis=0)
    else:
        sh = hr
    if dx != 0:
        xo = jax.lax.broadcasted_iota(jnp.int32, (1024, 1), 0) % 32
        valid = (xo + dx >= 0) & (xo + dx < 32)
        sh = jnp.where(valid, sh, 0.0)
    return sh


def _mega_body(x_ref, w1b_ref, b1b_ref, w2c_ref, b2c_ref, pq_ref,
               wr1_ref, wr2_ref, wp_ref, bp_ref,
               c_ref, ct_ref, csq_ref,
               zq_ref, idx_ref, loss_ref, perp_ref,
               cnt_ref, sse_ref):
    n = pl.program_id(0)

    pq = pq_ref[...]
    w1b = w1b_ref[...]
    b1b = b1b_ref[...]
    w2c = w2c_ref[...]
    b2c = b2c_ref[...]
    h = jnp.concatenate(
        [_conv_block(x_ref[0, :, 64 * i:64 * i + 64, :],
                     w1b, b1b, w2c, b2c, pq) for i in range(8)],
        axis=0)

    for _ in range(2):
        hr = jnp.maximum(h, 0.0)
        acc = jnp.zeros((1024, 64), jnp.float32)
        k = 0
        for ky in range(3):
            for kx in range(3):
                sh = _shift_tokens(hr, ky - 1, kx - 1)
                acc = acc + jnp.dot(sh, wr1_ref[k],
                                    preferred_element_type=jnp.float32)
                k += 1
        r = jnp.dot(jnp.maximum(acc, 0.0), wr2_ref[...],
                    preferred_element_type=jnp.float32)
        h = h + r

    h = jnp.maximum(h, 0.0)
    zf = jnp.dot(h, wp_ref[...],
                 preferred_element_type=jnp.float32) + bp_ref[...]

    scores = csq_ref[...] - 2.0 * jnp.dot(
        zf.astype(jnp.bfloat16), ct_ref[...].astype(jnp.bfloat16),
        preferred_element_type=jnp.float32)
    m = jnp.min(scores, axis=1, keepdims=True)
    jj = jax.lax.broadcasted_iota(jnp.int32, (1024, _NE), 1)
    idx = jnp.min(jnp.where(scores <= m, jj, _NE), axis=1, keepdims=True)

    onehot = (jj == idx).astype(jnp.float32)
    zq1 = jnp.dot(onehot, c_ref[...], preferred_element_type=jnp.float32)

    idx_ref[...] = idx[None]
    zq_ref[...] = jnp.transpose(zq1)[None]

    @pl.when(n == 0)
    def _():
        cnt_ref[...] = jnp.zeros_like(cnt_ref)
        sse_ref[0, 0] = 0.0

    cnt_ref[...] += jnp.sum(onehot, axis=0, keepdims=True)
    sse_ref[0, 0] += jnp.sum((zq1 - zf) ** 2)

    @pl.when(n == pl.num_programs(0) - 1)
    def _():
        total = sse_ref[0, 0]
        loss_ref[...] = jnp.full(
            (1, 1), (1.0 + _BETA) * total / (8.0 * _TOK * _ED), jnp.float32)
        e_mean = cnt_ref[...] / (8.0 * _TOK)
        ent = jnp.sum(e_mean * jnp.log(e_mean + 1e-10))
        perp_ref[...] = jnp.full((1, 1), jnp.exp(-ent), jnp.float32)


def kernel(x, conv1_w, conv1_b, conv2_w, conv2_b, res_w1, res_w2,
           preq_w, preq_b, codebook):
    f32 = jnp.float32

    w1t = conv1_w.transpose(0, 3, 1, 2)
    w1b = jnp.einsum('yb,odcr->yodcbr', jnp.eye(16, dtype=f32), w1t)
    w1b = w1b.reshape(1024, 768).astype(jnp.bfloat16)
    b1b = jnp.tile(conv1_b, 16).reshape(1024, 1)
    w2p = conv2_w.transpose(0, 3, 2, 1).reshape(128, 1024).astype(jnp.bfloat16)
    b2p = conv2_b.reshape(128, 1)
    ll = jnp.arange(128)
    pq = jnp.zeros((128, 128), jnp.bfloat16).at[
        ll, 32 * (ll % 4) + ll // 4].set(1.0)

    wr1 = res_w1.transpose(2, 3, 1, 0).reshape(9, 128, 64)
    wr2 = res_w2.reshape(128, 64).T
    wp = preq_w.reshape(64, 128).T
    bp = preq_b.reshape(1, 64)
    ct = codebook.T
    csq = jnp.sum(codebook ** 2, axis=1).reshape(1, _NE)

    zq, idx, loss, perp = pl.pallas_call(
        _mega_body,
        grid=(8,),
        in_specs=[
            pl.BlockSpec((1, 3, 512, 512), lambda n: (n, 0, 0, 0)),
            pl.BlockSpec((1024, 768), lambda n: (0, 0)),
            pl.BlockSpec((1024, 1), lambda n: (0, 0)),
            pl.BlockSpec((128, 1024), lambda n: (0, 0)),
            pl.BlockSpec((128, 1), lambda n: (0, 0)),
            pl.BlockSpec((128, 128), lambda n: (0, 0)),
            pl.BlockSpec((9, 128, 64), lambda n: (0, 0, 0)),
            pl.BlockSpec((64, 128), lambda n: (0, 0)),
            pl.BlockSpec((128, 64), lambda n: (0, 0)),
            pl.BlockSpec((1, 64), lambda n: (0, 0)),
            pl.BlockSpec((_NE, _ED), lambda n: (0, 0)),
            pl.BlockSpec((_ED, _NE), lambda n: (0, 0)),
            pl.BlockSpec((1, _NE), lambda n: (0, 0)),
        ],
        out_specs=[
            pl.BlockSpec((1, _ED, _TOK), lambda n: (n, 0, 0)),
            pl.BlockSpec((1, _TOK, 1), lambda n: (n, 0, 0)),
            pl.BlockSpec((1, 1), lambda n: (0, 0)),
            pl.BlockSpec((1, 1), lambda n: (0, 0)),
        ],
        out_shape=[
            jax.ShapeDtypeStruct((8, _ED, _TOK), f32),
            jax.ShapeDtypeStruct((8, _TOK, 1), jnp.int32),
            jax.ShapeDtypeStruct((1, 1), f32),
            jax.ShapeDtypeStruct((1, 1), f32),
        ],
        scratch_shapes=[
            pltpu.VMEM((1, _NE), f32),
            pltpu.SMEM((1, 1), f32),
        ],
    )(x, w1b, b1b, w2p, b2p, pq, wr1, wr2, wp, bp, codebook, ct, csq)

    z_q = zq.reshape(8, _ED, 32, 32)
    idx_out = idx.reshape(8 * _TOK, 1)
    return (loss[0, 0], z_q, perp[0, 0], codebook, idx_out)
